# baseline (speedup 1.0000x reference)
import functools

import jax
import jax.numpy as jnp
from jax import lax
from jax.experimental import pallas as pl
from jax.experimental.pallas import tpu as pltpu
from jax.experimental.pallas import tpu_sc as plsc

_N0, _N1, _N2, _N3 = 50000, 25000, 12500, 6250
_P0, _P1, _P2, _P3 = 50176, 25088, 12544, 6272

_CH = 128
_NTILE = 32


def _sconv_sc(n_pad, F, E_pad):
    e_tile = E_pad // _NTILE
    chunks = e_tile // _CH
    K = {4: 24, 8: 24, 16: 16, 32: 8, 64: 4, 128: 2}[F]
    rounds = chunks // K
    rounds -= rounds % 2
    tail = chunks - rounds * K
    assert tail < 2 * K
    pairs = rounds // 2
    share = n_pad // 16
    c_tile = e_tile // _CH
    mesh = plsc.VectorSubcoreMesh(core_axis_name="c", subcore_axis_name="s")

    @functools.partial(
        pl.kernel,
        out_type=jax.ShapeDtypeStruct((2, n_pad, F), jnp.float32),
        mesh=mesh,
        compiler_params=pltpu.CompilerParams(use_tc_tiling_on_sc=False),
        scratch_types=[
            pltpu.VMEM((2, K, _CH), jnp.int32),
            pltpu.VMEM((2, K, _CH), jnp.int32),
            pltpu.VMEM((2, K, _CH, F), jnp.float32),
            pltpu.VMEM_SHARED((n_pad, F), jnp.float32),
            pltpu.SemaphoreType.DMA,
            pltpu.SemaphoreType.DMA,
            pltpu.SemaphoreType.DMA,
            pltpu.SemaphoreType.DMA,
            pltpu.SemaphoreType.DMA,
            pltpu.SemaphoreType.DMA,
            pltpu.SemaphoreType.DMA,
            pltpu.SemaphoreType.DMA,
        ],
    )
    def k(tab, src2, dst2, zeros, out, srcv, dstv, rows, acc,
          si0, si1, di0, di1, g0, g1, s0, s1):
        semis = (si0, si1)
        semid = (di0, di1)
        semg = (g0, g1)
        sems = (s0, s1)
        c = lax.axis_index("c")
        s = lax.axis_index("s")
        pltpu.sync_copy(zeros.at[pl.ds(s * share, share)],
                        acc.at[pl.ds(s * share, share)])
        plsc.subcore_barrier()
        tile_c0 = (c * 16 + s) * c_tile

        def issue_src(rr, st):
            pltpu.async_copy(src2.at[pl.ds(tile_c0 + rr * K, K)],
                             srcv.at[st], semis[st])

        def issue_dst(rr, st):
            pltpu.async_copy(dst2.at[pl.ds(tile_c0 + rr * K, K)],
                             dstv.at[st], semid[st])

        def drain_src(st):
            pltpu.make_async_copy(src2.at[pl.ds(0, K)], srcv.at[st],
                                  semis[st]).wait()

        def drain_dst(st):
            pltpu.make_async_copy(dst2.at[pl.ds(0, K)], dstv.at[st],
                                  semid[st]).wait()

        def issue_gathers(st):
            for b in range(K):
                pltpu.async_copy(tab.at[srcv.at[st, b]], rows.at[st, b],
                                 semg[st])

        def drain_gathers(st):
            for b in range(K):
                pltpu.make_async_copy(tab.at[pl.ds(0, _CH)], rows.at[st, b],
                                      semg[st]).wait()

        def issue_scatters(st):
            for b in range(K):
                pltpu.async_copy(rows.at[st, b], acc.at[dstv.at[st, b]],
                                 sems[st], add=True)

        def drain_scatters(st):
            for b in range(K):
                pltpu.make_async_copy(rows.at[st, b], acc.at[dstv.at[st, b]],
                                      sems[st]).wait()

        if rounds > 0:
            issue_src(0, 0)
            issue_dst(0, 0)
            if rounds > 1:
                issue_src(1, 1)
            drain_src(0)
            issue_gathers(0)

            def round_tpl(rr, st):
                ot = 1 - st
                drain_gathers(st)

                @pl.when(rr + 2 < rounds)
                def _():
                    issue_src(rr + 2, st)
                drain_dst(st)
                issue_scatters(st)

                @pl.when(rr >= 1)
                def _():
                    drain_scatters(ot)

                @pl.when(rr + 1 < rounds)
                def _():
                    drain_src(ot)
                    issue_gathers(ot)
                    issue_dst(rr + 1, ot)

            def body(i, carry):
                round_tpl(2 * i, 0)
                round_tpl(2 * i + 1, 1)
                return carry

            lax.fori_loop(0, pairs, body, 0)
            drain_scatters((rounds - 1) % 2)

        if tail > 0:
            t0 = tail if tail <= K else K
            t1 = tail - t0
            pltpu.sync_copy(src2.at[pl.ds(tile_c0 + rounds * K, t0)],
                            srcv.at[0, pl.ds(0, t0)])
            pltpu.sync_copy(dst2.at[pl.ds(tile_c0 + rounds * K, t0)],
                            dstv.at[0, pl.ds(0, t0)])
            if t1 > 0:
                pltpu.sync_copy(src2.at[pl.ds(tile_c0 + rounds * K + t0, t1)],
                                srcv.at[1, pl.ds(0, t1)])
                pltpu.sync_copy(dst2.at[pl.ds(tile_c0 + rounds * K + t0, t1)],
                                dstv.at[1, pl.ds(0, t1)])
            gds = [pltpu.async_copy(tab.at[srcv.at[0, b]], rows.at[0, b], g0)
                   for b in range(t0)]
            gds += [pltpu.async_copy(tab.at[srcv.at[1, b]], rows.at[1, b], g1)
                    for b in range(t1)]
            for d in gds:
                d.wait()
            sds = [pltpu.async_copy(rows.at[0, b], acc.at[dstv.at[0, b]],
                                    s0, add=True) for b in range(t0)]
            sds += [pltpu.async_copy(rows.at[1, b], acc.at[dstv.at[1, b]],
                                     s1, add=True) for b in range(t1)]
            for d in sds:
                d.wait()

        plsc.subcore_barrier()
        pltpu.sync_copy(acc.at[pl.ds(s * share, share)],
                        out.at[c, pl.ds(s * share, share)])

    return k


def _mm(h, W, R):
    n, Fi = h.shape
    Fo = W.shape[1]

    def body(h_ref, w_ref, o_ref):
        o_ref[...] = jnp.dot(h_ref[...], w_ref[...],
                             preferred_element_type=jnp.float32)

    return pl.pallas_call(
        body,
        grid=(n // R,),
        in_specs=[pl.BlockSpec((R, Fi), lambda i: (i, 0)),
                  pl.BlockSpec((Fi, Fo), lambda i: (0, 0))],
        out_specs=pl.BlockSpec((R, Fo), lambda i: (i, 0)),
        out_shape=jax.ShapeDtypeStruct((n, Fo), jnp.float32),
    )(h, W)


def _stats(parts, n_real, R):
    n, F = parts[0].shape

    def body(*refs):
        i = pl.program_id(0)
        h = refs[0][...]
        for r in refs[1:-2]:
            h = h + r[...]
        s1_ref, s2_ref = refs[-2], refs[-1]
        rows = lax.broadcasted_iota(jnp.int32, (R, F), 0) + i * R
        h = jnp.where(rows < n_real, h, 0.0)
        s1 = jnp.sum(h, axis=0, keepdims=True)
        s2 = jnp.sum(h * h, axis=0, keepdims=True)

        @pl.when(i == 0)
        def _():
            s1_ref[...] = jnp.zeros_like(s1_ref)
            s2_ref[...] = jnp.zeros_like(s2_ref)

        s1_ref[...] += s1
        s2_ref[...] += s2

    return pl.pallas_call(
        body,
        grid=(n // R,),
        in_specs=[pl.BlockSpec((R, F), lambda i: (i, 0)) for _ in parts],
        out_specs=[pl.BlockSpec((1, F), lambda i: (0, 0)),
                   pl.BlockSpec((1, F), lambda i: (0, 0))],
        out_shape=[jax.ShapeDtypeStruct((1, F), jnp.float32),
                   jax.ShapeDtypeStruct((1, F), jnp.float32)],
    )(*parts)


def _bnapply(parts, s1, s2, g, b, n_real, R, res=None, Ws=(), head=None,
             want_act=False):
    n, F = parts[0].shape
    n_parts = len(parts)
    has_res = res is not None
    has_head = head is not None

    def body(*refs):
        it = iter(refs)
        h = next(it)[...]
        for _ in range(n_parts - 1):
            h = h + next(it)[...]
        s1v = next(it)[...]
        s2v = next(it)[...]
        gv = next(it)[...]
        bv = next(it)[...]
        mu = s1v / n_real
        var = s2v / n_real - mu * mu
        inv = lax.rsqrt(var + 1e-5)
        t = jax.nn.relu((h - mu) * inv * gv + bv)
        res_v = next(it)[...] if has_res else None
        if has_head:
            wo = next(it)[...]
            bo = next(it)[...]
        w_refs = [next(it) for _ in Ws]
        outs = list(it)
        oi = 0
        if want_act:
            outs[oi][...] = t
            oi += 1
        if has_head:
            outs[oi][...] = jax.nn.sigmoid(
                jnp.dot(t, wo, preferred_element_type=jnp.float32) + bo)
            oi += 1
        r = t + res_v if has_res else t
        for wr in w_refs:
            outs[oi][...] = jnp.dot(r, wr[...],
                                    preferred_element_type=jnp.float32)
            oi += 1

    in_arrays = list(parts) + [s1, s2, g, b]
    in_specs = [pl.BlockSpec((R, F), lambda i: (i, 0)) for _ in parts]
    in_specs += [pl.BlockSpec((1, F), lambda i: (0, 0))] * 4
    if has_res:
        in_arrays.append(res)
        in_specs.append(pl.BlockSpec((R, F), lambda i: (i, 0)))
    if has_head:
        wo, bo = head
        in_arrays += [wo, bo]
        in_specs += [pl.BlockSpec(wo.shape, lambda i: (0, 0)),
                     pl.BlockSpec((1, 8), lambda i: (0, 0))]
    for W in Ws:
        in_arrays.append(W)
        in_specs.append(pl.BlockSpec(W.shape, lambda i: (0, 0)))

    out_specs, out_shapes = [], []
    if want_act:
        out_specs.append(pl.BlockSpec((R, F), lambda i: (i, 0)))
        out_shapes.append(jax.ShapeDtypeStruct((n, F), jnp.float32))
    if has_head:
        out_specs.append(pl.BlockSpec((R, 8), lambda i: (i, 0)))
        out_shapes.append(jax.ShapeDtypeStruct((n, 8), jnp.float32))
    for W in Ws:
        out_specs.append(pl.BlockSpec((R, W.shape[1]), lambda i: (i, 0)))
        out_shapes.append(jax.ShapeDtypeStruct((n, W.shape[1]), jnp.float32))

    return pl.pallas_call(
        body,
        grid=(n // R,),
        in_specs=in_specs,
        out_specs=out_specs,
        out_shape=out_shapes,
    )(*in_arrays)


def _final(p0, p1, dfin, wo1, bo1, R):
    n = p0.shape[0]

    def body(p0_ref, p1_ref, d_ref, wo_ref, bo_ref, decf_ref, prob_ref):
        s = d_ref[...] + p0_ref[...] + p1_ref[...]
        decf = jax.nn.sigmoid(s)
        decf_ref[...] = decf
        prob_ref[...] = jax.nn.sigmoid(
            jnp.dot(decf, wo_ref[...], preferred_element_type=jnp.float32)
            + bo_ref[...])

    return pl.pallas_call(
        body,
        grid=(n // R,),
        in_specs=[pl.BlockSpec((R, 16), lambda i: (i, 0)),
                  pl.BlockSpec((R, 16), lambda i: (i, 0)),
                  pl.BlockSpec((R, 16), lambda i: (i, 0)),
                  pl.BlockSpec((16, 8), lambda i: (0, 0)),
                  pl.BlockSpec((1, 8), lambda i: (0, 0))],
        out_specs=[pl.BlockSpec((R, 16), lambda i: (i, 0)),
                   pl.BlockSpec((R, 8), lambda i: (i, 0))],
        out_shape=[jax.ShapeDtypeStruct((n, 16), jnp.float32),
                   jax.ShapeDtypeStruct((n, 8), jnp.float32)],
    )(p0, p1, dfin, wo1, bo1)


def _bnstage(parts, g, b, n_real, R, res=None, Ws=(), head=None,
             want_act=False):
    n, F = parts[0].shape
    n_parts = len(parts)
    has_res = res is not None
    has_head = head is not None

    def body(*refs):
        stat = refs[-1]
        refs = refs[:-1]
        p = pl.program_id(0)
        i = pl.program_id(1)
        it = iter(refs)
        h = next(it)[...]
        for _ in range(n_parts - 1):
            h = h + next(it)[...]
        gv = next(it)[...]
        bv = next(it)[...]
        res_v = next(it)[...] if has_res else None
        if has_head:
            wo = next(it)[...]
            bo = next(it)[...]
        w_refs = [next(it) for _ in Ws]
        outs = list(it)

        @pl.when(p == 0)
        def _():
            rows = lax.broadcasted_iota(jnp.int32, (R, F), 0) + i * R
            hm = jnp.where(rows < n_real, h, 0.0)
            s1 = jnp.sum(hm, axis=0, keepdims=True)
            s2 = jnp.sum(hm * hm, axis=0, keepdims=True)

            @pl.when(i == 0)
            def _():
                stat[...] = jnp.zeros_like(stat)

            stat[0:1, :] += s1
            stat[1:2, :] += s2

        @pl.when(p == 1)
        def _():
            mu = stat[0:1, :] / n_real
            var = stat[1:2, :] / n_real - mu * mu
            inv = lax.rsqrt(var + 1e-5)
            t = jax.nn.relu((h - mu) * inv * gv + bv)
            oi = 0
            if want_act:
                outs[oi][...] = t
                oi += 1
            if has_head:
                outs[oi][...] = jax.nn.sigmoid(
                    jnp.dot(t, wo, preferred_element_type=jnp.float32) + bo)
                oi += 1
            r = t + res_v if has_res else t
            for wr in w_refs:
                outs[oi][...] = jnp.dot(r, wr[...],
                                        preferred_element_type=jnp.float32)
                oi += 1

    in_arrays = list(parts) + [g, b]
    in_specs = [pl.BlockSpec((R, F), lambda p, i: (i, 0)) for _ in parts]
    in_specs += [pl.BlockSpec((1, F), lambda p, i: (0, 0))] * 2
    if has_res:
        in_arrays.append(res)
        in_specs.append(pl.BlockSpec((R, F), lambda p, i: (i, 0)))
    if has_head:
        wo, bo = head
        in_arrays += [wo, bo]
        in_specs += [pl.BlockSpec(wo.shape, lambda p, i: (0, 0)),
                     pl.BlockSpec((1, 8), lambda p, i: (0, 0))]
    for W in Ws:
        in_arrays.append(W)
        in_specs.append(pl.BlockSpec(W.shape, lambda p, i: (0, 0)))

    out_specs, out_shapes = [], []
    if want_act:
        out_specs.append(pl.BlockSpec((R, F), lambda p, i: (i, 0)))
        out_shapes.append(jax.ShapeDtypeStruct((n, F), jnp.float32))
    if has_head:
        out_specs.append(pl.BlockSpec((R, 8), lambda p, i: (i, 0)))
        out_shapes.append(jax.ShapeDtypeStruct((n, 8), jnp.float32))
    for W in Ws:
        out_specs.append(pl.BlockSpec((R, W.shape[1]), lambda p, i: (i, 0)))
        out_shapes.append(jax.ShapeDtypeStruct((n, W.shape[1]), jnp.float32))

    return pl.pallas_call(
        body,
        grid=(2, n // R),
        in_specs=in_specs,
        out_specs=out_specs,
        out_shape=out_shapes,
        scratch_shapes=[pltpu.VMEM((2, F), jnp.float32)],
    )(*in_arrays)


def _mm2(h, W1, W2, R):
    n, Fi = h.shape

    def body(h_ref, w1_ref, w2_ref, o1_ref, o2_ref):
        hv = h_ref[...]
        o1_ref[...] = jnp.dot(hv, w1_ref[...],
                              preferred_element_type=jnp.float32)
        o2_ref[...] = jnp.dot(hv, w2_ref[...],
                              preferred_element_type=jnp.float32)

    return pl.pallas_call(
        body,
        grid=(n // R,),
        in_specs=[pl.BlockSpec((R, Fi), lambda i: (i, 0)),
                  pl.BlockSpec(W1.shape, lambda i: (0, 0)),
                  pl.BlockSpec(W2.shape, lambda i: (0, 0))],
        out_specs=[pl.BlockSpec((R, W1.shape[1]), lambda i: (i, 0)),
                   pl.BlockSpec((R, W2.shape[1]), lambda i: (i, 0))],
        out_shape=[jax.ShapeDtypeStruct((n, W1.shape[1]), jnp.float32),
                   jax.ShapeDtypeStruct((n, W2.shape[1]), jnp.float32)],
    )(h, W1, W2)


def _pad_edges(src, dst, n_pad_dst):
    e = src.shape[0]
    e_pad = -(-e // (_NTILE * _CH)) * (_NTILE * _CH)
    if e_pad != e:
        src = jnp.concatenate(
            [src, jnp.zeros((e_pad - e,), jnp.int32)])
        dst = jnp.concatenate(
            [dst, jnp.full((e_pad - e,), n_pad_dst - 1, jnp.int32)])
    return src.reshape(-1, _CH), dst.reshape(-1, _CH), e_pad


def _pad_w(W, cols):
    Fi, Fo = W.shape
    return jnp.pad(W, ((0, 0), (0, cols - Fo)))


def _row2d(v, width):
    v = jnp.asarray(v, jnp.float32).reshape(1, -1)
    return jnp.pad(v, ((0, 0), (0, width - v.shape[1])))


def kernel(x, ei0_src, ei0_dst, e2_src, e2_dst, e3_src, e3_dst, e4_src,
           e4_dst, params):
    p = params
    f32 = jnp.float32

    x_p = jnp.pad(x.astype(f32), ((0, _P0 - _N0), (0, 0)))
    z0_16 = jnp.zeros((_P0, 16), f32)
    z1_32 = jnp.zeros((_P1, 32), f32)
    z2_64 = jnp.zeros((_P2, 64), f32)
    z3_128 = jnp.zeros((_P3, 128), f32)

    ei_s, ei_d, E0p = _pad_edges(ei0_src, ei0_dst, _P0)
    e2_s, e2_d, E2p = _pad_edges(e2_src, e2_dst, _P1)
    e3_s, e3_d, E3p = _pad_edges(e3_src, e3_dst, _P2)
    e4_s, e4_d, E4p = _pad_edges(e4_src, e4_dst, _P3)
    e2r_s, e2r_d, _ = _pad_edges(e2_dst, e2_src, _P0)
    e3r_s, e3r_d, _ = _pad_edges(e3_dst, e3_src, _P1)
    e4r_s, e4r_d, _ = _pad_edges(e4_dst, e4_src, _P2)

    g1 = _row2d(p["g1"], 16); b1 = _row2d(p["b1"], 16)
    g2 = _row2d(p["g2"], 32); b2 = _row2d(p["b2"], 32)
    g3 = _row2d(p["g3"], 64); b3 = _row2d(p["b3"], 64)
    g4 = _row2d(p["g4"], 128); b4 = _row2d(p["b4"], 128)
    gd4 = _row2d(p["gd4"], 64); bd4 = _row2d(p["bd4"], 64)
    gd3 = _row2d(p["gd3"], 32); bd3 = _row2d(p["bd3"], 32)
    gd2 = _row2d(p["gd2"], 16); bd2 = _row2d(p["bd2"], 16)
    wo4 = _pad_w(p["wo4"], 8); bo4 = _row2d(p["bo4"], 8)
    wo3 = _pad_w(p["wo3"], 8); bo3 = _row2d(p["bo3"], 8)
    wo2 = _pad_w(p["wo2"], 8); bo2 = _row2d(p["bo2"], 8)
    wo1 = jnp.pad(_pad_w(p["wo1"], 8), ((0, 16 - 3), (0, 0)))
    bo1 = _row2d(p["bo1"], 8)
    wd1n = _pad_w(p["Wd1n"], 16)
    wd1s = _pad_w(p["Wd1s"], 16)

    tab0, d0 = _mm2(x_p, p["W1n"], p["W1s"], 512)
    P = _sconv_sc(_P0, 16, E0p)(tab0, ei_s, ei_d, z0_16)
    enc0, tab1 = _bnstage([P[0], P[1], d0], g1, b1, _N0, 512,
                          Ws=(p["W2"],), want_act=True)
    P = _sconv_sc(_P1, 32, E2p)(tab1, e2_s, e2_d, z1_32)
    enc1, tab2 = _bnstage([P[0], P[1]], g2, b2, _N1, 512,
                          Ws=(p["W3"],), want_act=True)
    P = _sconv_sc(_P2, 64, E3p)(tab2, e3_s, e3_d, z2_64)
    enc2, tab3 = _bnstage([P[0], P[1]], g3, b3, _N2, 256,
                          Ws=(p["W4"],), want_act=True)
    P = _sconv_sc(_P3, 128, E4p)(tab3, e4_s, e4_d, z3_128)
    tab4, = _bnstage([P[0], P[1]], g4, b4, _N3, 128, Ws=(p["Wd4"],))
    P = _sconv_sc(_P2, 64, E4p)(tab4, e4r_s, e4r_d, z2_64)
    prob4, tab5 = _bnstage([P[0], P[1]], gd4, bd4, _N2, 256,
                           res=enc2, Ws=(p["Wd3"],), head=(wo4, bo4))
    P = _sconv_sc(_P1, 32, E3p)(tab5, e3r_s, e3r_d, z1_32)
    prob3, tab6 = _bnstage([P[0], P[1]], gd3, bd3, _N1, 512,
                           res=enc1, Ws=(p["Wd2"],), head=(wo3, bo3))
    P = _sconv_sc(_P0, 16, E2p)(tab6, e2r_s, e2r_d, z0_16)
    prob2, tab7, dfin = _bnstage([P[0], P[1]], gd2, bd2, _N0, 512,
                                 res=enc0, Ws=(wd1n, wd1s),
                                 head=(wo2, bo2))
    P = _sconv_sc(_P0, 16, E0p)(tab7, ei_s, ei_d, z0_16)
    decf4, prob1 = _final(P[0], P[1], dfin, wo1, bo1, 512)

    return (decf4[:_N0, :3], prob4[:_N2, :1], prob3[:_N1, :1],
            prob2[:_N0, :1], prob1[:_N0, :1])


# speedup vs baseline: 1.1163x; 1.1163x over previous
import functools

import jax
import jax.numpy as jnp
from jax import lax
from jax.experimental import pallas as pl
from jax.experimental.pallas import tpu as pltpu
from jax.experimental.pallas import tpu_sc as plsc

_N0, _N1, _N2, _N3 = 50000, 25000, 12500, 6250
_P0, _P1, _P2, _P3 = 50176, 25088, 12544, 6272

_CH = 128
_NTILE = 32


def _sconv_sc(n_pad, F, E_pad):
    e_tile = E_pad // _NTILE
    chunks = e_tile // _CH
    K = {16: 16, 32: 8, 64: 4, 128: 2}[F]
    rounds = chunks // K
    rounds -= rounds % 2
    tail = chunks - rounds * K
    assert tail < 2 * K
    pairs = rounds // 2
    share = n_pad // 16
    mesh = plsc.VectorSubcoreMesh(core_axis_name="c", subcore_axis_name="s")

    @functools.partial(
        pl.kernel,
        out_type=jax.ShapeDtypeStruct((2, n_pad, F), jnp.float32),
        mesh=mesh,
        compiler_params=pltpu.CompilerParams(use_tc_tiling_on_sc=False),
        scratch_types=[
            pltpu.VMEM((2, K, _CH), jnp.int32),
            pltpu.VMEM((2, K, _CH), jnp.int32),
            pltpu.VMEM((2, K, _CH, F), jnp.float32),
            pltpu.VMEM_SHARED((n_pad, F), jnp.float32),
            pltpu.SemaphoreType.DMA,
            pltpu.SemaphoreType.DMA,
            pltpu.SemaphoreType.DMA,
            pltpu.SemaphoreType.DMA,
            pltpu.SemaphoreType.DMA,
            pltpu.SemaphoreType.DMA,
            pltpu.SemaphoreType.DMA,
            pltpu.SemaphoreType.DMA,
        ],
    )
    def k(tab, src, dst, zeros, out, srcv, dstv, rows, acc,
          si0, si1, di0, di1, g0, g1, s0, s1):
        semis = (si0, si1)
        semid = (di0, di1)
        semg = (g0, g1)
        sems = (s0, s1)
        c = lax.axis_index("c")
        s = lax.axis_index("s")
        pltpu.sync_copy(zeros.at[pl.ds(s * share, share)],
                        acc.at[pl.ds(s * share, share)])
        plsc.subcore_barrier()
        tile_e0 = (c * 16 + s) * e_tile

        def issue_src(rr, st):
            for b in range(K):
                pltpu.async_copy(
                    src.at[pl.ds(tile_e0 + (rr * K + b) * _CH, _CH)],
                    srcv.at[st, b], semis[st])

        def issue_dst(rr, st):
            for b in range(K):
                pltpu.async_copy(
                    dst.at[pl.ds(tile_e0 + (rr * K + b) * _CH, _CH)],
                    dstv.at[st, b], semid[st])

        def drain_src(st):
            for b in range(K):
                pltpu.make_async_copy(src.at[pl.ds(0, _CH)],
                                      srcv.at[st, b], semis[st]).wait()

        def drain_dst(st):
            for b in range(K):
                pltpu.make_async_copy(dst.at[pl.ds(0, _CH)],
                                      dstv.at[st, b], semid[st]).wait()

        def issue_gathers(st):
            for b in range(K):
                pltpu.async_copy(tab.at[srcv.at[st, b]], rows.at[st, b],
                                 semg[st])

        def drain_gathers(st):
            for b in range(K):
                pltpu.make_async_copy(tab.at[pl.ds(0, _CH)], rows.at[st, b],
                                      semg[st]).wait()

        def issue_scatters(st):
            for b in range(K):
                pltpu.async_copy(rows.at[st, b], acc.at[dstv.at[st, b]],
                                 sems[st], add=True)

        def drain_scatters(st):
            for b in range(K):
                pltpu.make_async_copy(rows.at[st, b], acc.at[dstv.at[st, b]],
                                      sems[st]).wait()

        if rounds > 0:
            issue_src(0, 0)
            issue_dst(0, 0)
            if rounds > 1:
                issue_src(1, 1)
            drain_src(0)
            issue_gathers(0)

            def round_tpl(rr, st):
                ot = 1 - st
                drain_gathers(st)

                @pl.when(rr + 2 < rounds)
                def _():
                    issue_src(rr + 2, st)
                drain_dst(st)
                issue_scatters(st)

                @pl.when(rr >= 1)
                def _():
                    drain_scatters(ot)

                @pl.when(rr + 1 < rounds)
                def _():
                    drain_src(ot)
                    issue_gathers(ot)
                    issue_dst(rr + 1, ot)

            def body(i, carry):
                round_tpl(2 * i, 0)
                round_tpl(2 * i + 1, 1)
                return carry

            lax.fori_loop(0, pairs, body, 0)
            drain_scatters((rounds - 1) % 2)

        if tail > 0:
            t0 = tail if tail <= K else K
            t1 = tail - t0
            for b in range(t0):
                pltpu.sync_copy(
                    src.at[pl.ds(tile_e0 + (rounds * K + b) * _CH, _CH)],
                    srcv.at[0, b])
                pltpu.sync_copy(
                    dst.at[pl.ds(tile_e0 + (rounds * K + b) * _CH, _CH)],
                    dstv.at[0, b])
            for b in range(t1):
                pltpu.sync_copy(
                    src.at[pl.ds(tile_e0 + (rounds * K + t0 + b) * _CH, _CH)],
                    srcv.at[1, b])
                pltpu.sync_copy(
                    dst.at[pl.ds(tile_e0 + (rounds * K + t0 + b) * _CH, _CH)],
                    dstv.at[1, b])
            gds = [pltpu.async_copy(tab.at[srcv.at[0, b]], rows.at[0, b], g0)
                   for b in range(t0)]
            gds += [pltpu.async_copy(tab.at[srcv.at[1, b]], rows.at[1, b], g1)
                    for b in range(t1)]
            for d in gds:
                d.wait()
            sds = [pltpu.async_copy(rows.at[0, b], acc.at[dstv.at[0, b]],
                                    s0, add=True) for b in range(t0)]
            sds += [pltpu.async_copy(rows.at[1, b], acc.at[dstv.at[1, b]],
                                     s1, add=True) for b in range(t1)]
            for d in sds:
                d.wait()

        plsc.subcore_barrier()
        pltpu.sync_copy(acc.at[pl.ds(s * share, share)],
                        out.at[c, pl.ds(s * share, share)])

    return k


def _mm(h, W, R):
    n, Fi = h.shape
    Fo = W.shape[1]

    def body(h_ref, w_ref, o_ref):
        o_ref[...] = jnp.dot(h_ref[...], w_ref[...],
                             preferred_element_type=jnp.float32)

    return pl.pallas_call(
        body,
        grid=(n // R,),
        in_specs=[pl.BlockSpec((R, Fi), lambda i: (i, 0)),
                  pl.BlockSpec((Fi, Fo), lambda i: (0, 0))],
        out_specs=pl.BlockSpec((R, Fo), lambda i: (i, 0)),
        out_shape=jax.ShapeDtypeStruct((n, Fo), jnp.float32),
    )(h, W)


def _part_spec(arr, lead, R, F):
    if lead is None:
        return pl.BlockSpec((R, F), lambda i: (i, 0))
    return pl.BlockSpec((1, R, F), lambda i, l=lead: (l, i, 0))


def _part_val(ref):
    v = ref[...]
    return v[0] if v.ndim == 3 else v


def _stats(parts, n_real, R):
    F = parts[0][0].shape[-1]
    n = parts[0][0].shape[-2]
    nparts = len(parts)

    def body(*refs):
        i = pl.program_id(0)
        h = _part_val(refs[0])
        for r in refs[1:nparts]:
            h = h + _part_val(r)
        s1_ref, s2_ref = refs[-2], refs[-1]
        rows = lax.broadcasted_iota(jnp.int32, (R, F), 0) + i * R
        h = jnp.where(rows < n_real, h, 0.0)
        s1 = jnp.sum(h, axis=0, keepdims=True)
        s2 = jnp.sum(h * h, axis=0, keepdims=True)

        @pl.when(i == 0)
        def _():
            s1_ref[...] = jnp.zeros_like(s1_ref)
            s2_ref[...] = jnp.zeros_like(s2_ref)

        s1_ref[...] += s1
        s2_ref[...] += s2

    return pl.pallas_call(
        body,
        grid=(n // R,),
        in_specs=[_part_spec(a, l, R, F) for a, l in parts],
        out_specs=[pl.BlockSpec((1, F), lambda i: (0, 0)),
                   pl.BlockSpec((1, F), lambda i: (0, 0))],
        out_shape=[jax.ShapeDtypeStruct((1, F), jnp.float32),
                   jax.ShapeDtypeStruct((1, F), jnp.float32)],
    )(*[a for a, _ in parts])


def _bnapply(parts, s1, s2, g, b, n_real, R, res=None, Ws=(), head=None,
             want_act=False):
    F = parts[0][0].shape[-1]
    n = parts[0][0].shape[-2]
    n_parts = len(parts)
    has_res = res is not None
    has_head = head is not None

    def body(*refs):
        it = iter(refs)
        h = _part_val(next(it))
        for _ in range(n_parts - 1):
            h = h + _part_val(next(it))
        s1v = next(it)[...]
        s2v = next(it)[...]
        gv = next(it)[...]
        bv = next(it)[...]
        mu = s1v / n_real
        var = s2v / n_real - mu * mu
        inv = lax.rsqrt(var + 1e-5)
        t = jax.nn.relu((h - mu) * inv * gv + bv)
        res_v = next(it)[...] if has_res else None
        if has_head:
            wo = next(it)[...]
            bo = next(it)[...]
        w_refs = [next(it) for _ in Ws]
        outs = list(it)
        oi = 0
        if want_act:
            outs[oi][...] = t
            oi += 1
        if has_head:
            outs[oi][...] = jax.nn.sigmoid(
                jnp.dot(t, wo, preferred_element_type=jnp.float32) + bo)
            oi += 1
        r = t + res_v if has_res else t
        for wr in w_refs:
            outs[oi][...] = jnp.dot(r, wr[...],
                                    preferred_element_type=jnp.float32)
            oi += 1

    in_arrays = [a for a, _ in parts] + [s1, s2, g, b]
    in_specs = [_part_spec(a, l, R, F) for a, l in parts]
    in_specs += [pl.BlockSpec((1, F), lambda i: (0, 0))] * 4
    if has_res:
        in_arrays.append(res)
        in_specs.append(pl.BlockSpec((R, F), lambda i: (i, 0)))
    if has_head:
        wo, bo = head
        in_arrays += [wo, bo]
        in_specs += [pl.BlockSpec(wo.shape, lambda i: (0, 0)),
                     pl.BlockSpec((1, 8), lambda i: (0, 0))]
    for W in Ws:
        in_arrays.append(W)
        in_specs.append(pl.BlockSpec(W.shape, lambda i: (0, 0)))

    out_specs, out_shapes = [], []
    if want_act:
        out_specs.append(pl.BlockSpec((R, F), lambda i: (i, 0)))
        out_shapes.append(jax.ShapeDtypeStruct((n, F), jnp.float32))
    if has_head:
        out_specs.append(pl.BlockSpec((R, 8), lambda i: (i, 0)))
        out_shapes.append(jax.ShapeDtypeStruct((n, 8), jnp.float32))
    for W in Ws:
        out_specs.append(pl.BlockSpec((R, W.shape[1]), lambda i: (i, 0)))
        out_shapes.append(jax.ShapeDtypeStruct((n, W.shape[1]), jnp.float32))

    return pl.pallas_call(
        body,
        grid=(n // R,),
        in_specs=in_specs,
        out_specs=out_specs,
        out_shape=out_shapes,
    )(*in_arrays)


def _final(P, dfin, wo1, bo1, R):
    n = P.shape[1]

    def body(p0_ref, p1_ref, d_ref, wo_ref, bo_ref, decf_ref, prob_ref):
        s = d_ref[...] + p0_ref[...][0] + p1_ref[...][0]
        decf = jax.nn.sigmoid(s)
        decf_ref[...] = decf
        prob_ref[...] = jax.nn.sigmoid(
            jnp.dot(decf, wo_ref[...], preferred_element_type=jnp.float32)
            + bo_ref[...])

    return pl.pallas_call(
        body,
        grid=(n // R,),
        in_specs=[pl.BlockSpec((1, R, 16), lambda i: (0, i, 0)),
                  pl.BlockSpec((1, R, 16), lambda i: (1, i, 0)),
                  pl.BlockSpec((R, 16), lambda i: (i, 0)),
                  pl.BlockSpec((16, 8), lambda i: (0, 0)),
                  pl.BlockSpec((1, 8), lambda i: (0, 0))],
        out_specs=[pl.BlockSpec((R, 16), lambda i: (i, 0)),
                   pl.BlockSpec((R, 8), lambda i: (i, 0))],
        out_shape=[jax.ShapeDtypeStruct((n, 16), jnp.float32),
                   jax.ShapeDtypeStruct((n, 8), jnp.float32)],
    )(P, P, dfin, wo1, bo1)


def _bnstage(parts, g, b, n_real, R, res=None, Ws=(), head=None,
             want_act=False):
    n, F = parts[0].shape
    n_parts = len(parts)
    has_res = res is not None
    has_head = head is not None

    def body(*refs):
        stat = refs[-1]
        refs = refs[:-1]
        p = pl.program_id(0)
        i = pl.program_id(1)
        it = iter(refs)
        h = next(it)[...]
        for _ in range(n_parts - 1):
            h = h + next(it)[...]
        gv = next(it)[...]
        bv = next(it)[...]
        res_v = next(it)[...] if has_res else None
        if has_head:
            wo = next(it)[...]
            bo = next(it)[...]
        w_refs = [next(it) for _ in Ws]
        outs = list(it)

        @pl.when(p == 0)
        def _():
            rows = lax.broadcasted_iota(jnp.int32, (R, F), 0) + i * R
            hm = jnp.where(rows < n_real, h, 0.0)
            s1 = jnp.sum(hm, axis=0, keepdims=True)
            s2 = jnp.sum(hm * hm, axis=0, keepdims=True)

            @pl.when(i == 0)
            def _():
                stat[...] = jnp.zeros_like(stat)

            stat[0:1, :] += s1
            stat[1:2, :] += s2

        @pl.when(p == 1)
        def _():
            mu = stat[0:1, :] / n_real
            var = stat[1:2, :] / n_real - mu * mu
            inv = lax.rsqrt(var + 1e-5)
            t = jax.nn.relu((h - mu) * inv * gv + bv)
            oi = 0
            if want_act:
                outs[oi][...] = t
                oi += 1
            if has_head:
                outs[oi][...] = jax.nn.sigmoid(
                    jnp.dot(t, wo, preferred_element_type=jnp.float32) + bo)
                oi += 1
            r = t + res_v if has_res else t
            for wr in w_refs:
                outs[oi][...] = jnp.dot(r, wr[...],
                                        preferred_element_type=jnp.float32)
                oi += 1

    in_arrays = list(parts) + [g, b]
    in_specs = [pl.BlockSpec((R, F), lambda p, i: (i, 0)) for _ in parts]
    in_specs += [pl.BlockSpec((1, F), lambda p, i: (0, 0))] * 2
    if has_res:
        in_arrays.append(res)
        in_specs.append(pl.BlockSpec((R, F), lambda p, i: (i, 0)))
    if has_head:
        wo, bo = head
        in_arrays += [wo, bo]
        in_specs += [pl.BlockSpec(wo.shape, lambda p, i: (0, 0)),
                     pl.BlockSpec((1, 8), lambda p, i: (0, 0))]
    for W in Ws:
        in_arrays.append(W)
        in_specs.append(pl.BlockSpec(W.shape, lambda p, i: (0, 0)))

    out_specs, out_shapes = [], []
    if want_act:
        out_specs.append(pl.BlockSpec((R, F), lambda p, i: (i, 0)))
        out_shapes.append(jax.ShapeDtypeStruct((n, F), jnp.float32))
    if has_head:
        out_specs.append(pl.BlockSpec((R, 8), lambda p, i: (i, 0)))
        out_shapes.append(jax.ShapeDtypeStruct((n, 8), jnp.float32))
    for W in Ws:
        out_specs.append(pl.BlockSpec((R, W.shape[1]), lambda p, i: (i, 0)))
        out_shapes.append(jax.ShapeDtypeStruct((n, W.shape[1]), jnp.float32))

    return pl.pallas_call(
        body,
        grid=(2, n // R),
        in_specs=in_specs,
        out_specs=out_specs,
        out_shape=out_shapes,
        scratch_shapes=[pltpu.VMEM((2, F), jnp.float32)],
    )(*in_arrays)


def _mm2(h, W1, W2, R):
    n, Fi = h.shape

    def body(h_ref, w1_ref, w2_ref, o1_ref, o2_ref):
        hv = h_ref[...]
        o1_ref[...] = jnp.dot(hv, w1_ref[...],
                              preferred_element_type=jnp.float32)
        o2_ref[...] = jnp.dot(hv, w2_ref[...],
                              preferred_element_type=jnp.float32)

    return pl.pallas_call(
        body,
        grid=(n // R,),
        in_specs=[pl.BlockSpec((R, Fi), lambda i: (i, 0)),
                  pl.BlockSpec(W1.shape, lambda i: (0, 0)),
                  pl.BlockSpec(W2.shape, lambda i: (0, 0))],
        out_specs=[pl.BlockSpec((R, W1.shape[1]), lambda i: (i, 0)),
                   pl.BlockSpec((R, W2.shape[1]), lambda i: (i, 0))],
        out_shape=[jax.ShapeDtypeStruct((n, W1.shape[1]), jnp.float32),
                   jax.ShapeDtypeStruct((n, W2.shape[1]), jnp.float32)],
    )(h, W1, W2)


def _pad_edges(src, dst, n_pad_dst):
    e = src.shape[0]
    e_pad = -(-e // (_NTILE * _CH)) * (_NTILE * _CH)
    if e_pad != e:
        src = jnp.concatenate(
            [src, jnp.zeros((e_pad - e,), jnp.int32)])
        dst = jnp.concatenate(
            [dst, jnp.full((e_pad - e,), n_pad_dst - 1, jnp.int32)])
    return src, dst, e_pad


def _pad_w(W, cols):
    Fi, Fo = W.shape
    return jnp.pad(W, ((0, 0), (0, cols - Fo)))


def _row2d(v, width):
    v = jnp.asarray(v, jnp.float32).reshape(1, -1)
    return jnp.pad(v, ((0, 0), (0, width - v.shape[1])))


def kernel(x, ei0_src, ei0_dst, e2_src, e2_dst, e3_src, e3_dst, e4_src,
           e4_dst, params):
    p = params
    f32 = jnp.float32

    x_p = jnp.pad(x.astype(f32), ((0, _P0 - _N0), (0, 0)))
    z0_16 = jnp.zeros((_P0, 16), f32)
    z1_32 = jnp.zeros((_P1, 32), f32)
    z2_64 = jnp.zeros((_P2, 64), f32)
    z3_128 = jnp.zeros((_P3, 128), f32)

    ei_s, ei_d, E0p = _pad_edges(ei0_src, ei0_dst, _P0)
    e2_s, e2_d, E2p = _pad_edges(e2_src, e2_dst, _P1)
    e3_s, e3_d, E3p = _pad_edges(e3_src, e3_dst, _P2)
    e4_s, e4_d, E4p = _pad_edges(e4_src, e4_dst, _P3)
    e2r_s, e2r_d, _ = _pad_edges(e2_dst, e2_src, _P0)
    e3r_s, e3r_d, _ = _pad_edges(e3_dst, e3_src, _P1)
    e4r_s, e4r_d, _ = _pad_edges(e4_dst, e4_src, _P2)

    g1 = _row2d(p["g1"], 16); b1 = _row2d(p["b1"], 16)
    g2 = _row2d(p["g2"], 32); b2 = _row2d(p["b2"], 32)
    g3 = _row2d(p["g3"], 64); b3 = _row2d(p["b3"], 64)
    g4 = _row2d(p["g4"], 128); b4 = _row2d(p["b4"], 128)
    gd4 = _row2d(p["gd4"], 64); bd4 = _row2d(p["bd4"], 64)
    gd3 = _row2d(p["gd3"], 32); bd3 = _row2d(p["bd3"], 32)
    gd2 = _row2d(p["gd2"], 16); bd2 = _row2d(p["bd2"], 16)
    wo4 = _pad_w(p["wo4"], 8); bo4 = _row2d(p["bo4"], 8)
    wo3 = _pad_w(p["wo3"], 8); bo3 = _row2d(p["bo3"], 8)
    wo2 = _pad_w(p["wo2"], 8); bo2 = _row2d(p["bo2"], 8)
    wo1 = jnp.pad(_pad_w(p["wo1"], 8), ((0, 16 - 3), (0, 0)))
    bo1 = _row2d(p["bo1"], 8)
    wd1n = _pad_w(p["Wd1n"], 16)
    wd1s = _pad_w(p["Wd1s"], 16)

    tab0, d0 = _mm2(x_p, p["W1n"], p["W1s"], 512)
    P = _sconv_sc(_P0, 16, E0p)(tab0, ei_s, ei_d, z0_16)
    parts = [(P, 0), (P, 1), (d0, None)]
    s1, s2 = _stats(parts, _N0, 512)
    enc0, tab1 = _bnapply(parts, s1, s2, g1, b1, _N0, 512,
                          Ws=(p["W2"],), want_act=True)
    P = _sconv_sc(_P1, 32, E2p)(tab1, e2_s, e2_d, z1_32)
    parts = [(P, 0), (P, 1)]
    s1, s2 = _stats(parts, _N1, 512)
    enc1, tab2 = _bnapply(parts, s1, s2, g2, b2, _N1, 512,
                          Ws=(p["W3"],), want_act=True)
    P = _sconv_sc(_P2, 64, E3p)(tab2, e3_s, e3_d, z2_64)
    parts = [(P, 0), (P, 1)]
    s1, s2 = _stats(parts, _N2, 256)
    enc2, tab3 = _bnapply(parts, s1, s2, g3, b3, _N2, 256,
                          Ws=(p["W4"],), want_act=True)
    P = _sconv_sc(_P3, 128, E4p)(tab3, e4_s, e4_d, z3_128)
    parts = [(P, 0), (P, 1)]
    s1, s2 = _stats(parts, _N3, 128)
    tab4, = _bnapply(parts, s1, s2, g4, b4, _N3, 128, Ws=(p["Wd4"],))
    P = _sconv_sc(_P2, 64, E4p)(tab4, e4r_s, e4r_d, z2_64)
    parts = [(P, 0), (P, 1)]
    s1, s2 = _stats(parts, _N2, 256)
    prob4, tab5 = _bnapply(parts, s1, s2, gd4, bd4, _N2, 256,
                           res=enc2, Ws=(p["Wd3"],), head=(wo4, bo4))
    P = _sconv_sc(_P1, 32, E3p)(tab5, e3r_s, e3r_d, z1_32)
    parts = [(P, 0), (P, 1)]
    s1, s2 = _stats(parts, _N1, 512)
    prob3, tab6 = _bnapply(parts, s1, s2, gd3, bd3, _N1, 512,
                           res=enc1, Ws=(p["Wd2"],), head=(wo3, bo3))
    P = _sconv_sc(_P0, 16, E2p)(tab6, e2r_s, e2r_d, z0_16)
    parts = [(P, 0), (P, 1)]
    s1, s2 = _stats(parts, _N0, 512)
    prob2, tab7, dfin = _bnapply(parts, s1, s2, gd2, bd2, _N0, 512,
                                 res=enc0, Ws=(wd1n, wd1s),
                                 head=(wo2, bo2))
    P = _sconv_sc(_P0, 16, E0p)(tab7, ei_s, ei_d, z0_16)
    decf4, prob1 = _final(P, dfin, wo1, bo1, 512)

    return (decf4[:_N0, :3], prob4[:_N2, :1], prob3[:_N1, :1],
            prob2[:_N0, :1], prob1[:_N0, :1])


# speedup vs baseline: 1.2425x; 1.1130x over previous
import functools

import jax
import jax.numpy as jnp
from jax import lax
from jax.experimental import pallas as pl
from jax.experimental.pallas import tpu as pltpu
from jax.experimental.pallas import tpu_sc as plsc

_N0, _N1, _N2, _N3 = 50000, 25000, 12500, 6250
_P0, _P1, _P2, _P3 = 50176, 25088, 12544, 6272

_CH = 128
_NTILE = 32


def _sconv_sc(n_pad, F, E_pad):
    e_tile = E_pad // _NTILE
    chunks = e_tile // _CH
    K = {16: 16, 32: 8, 64: 4, 128: 2}[F]
    rounds = chunks // K
    rounds -= rounds % 2
    tail = chunks - rounds * K
    assert tail < 2 * K
    pairs = rounds // 2
    share = n_pad // 16
    mesh = plsc.VectorSubcoreMesh(core_axis_name="c", subcore_axis_name="s")

    @functools.partial(
        pl.kernel,
        out_type=jax.ShapeDtypeStruct((2, n_pad, F), jnp.float32),
        mesh=mesh,
        compiler_params=pltpu.CompilerParams(use_tc_tiling_on_sc=False),
        scratch_types=[
            pltpu.VMEM((2, K, _CH), jnp.int32),
            pltpu.VMEM((2, K, _CH), jnp.int32),
            pltpu.VMEM((2, K, _CH, F), jnp.float32),
            pltpu.VMEM_SHARED((n_pad, F), jnp.float32),
            pltpu.SemaphoreType.DMA,
            pltpu.SemaphoreType.DMA,
            pltpu.SemaphoreType.DMA,
            pltpu.SemaphoreType.DMA,
            pltpu.SemaphoreType.DMA,
            pltpu.SemaphoreType.DMA,
            pltpu.SemaphoreType.DMA,
            pltpu.SemaphoreType.DMA,
        ],
    )
    def k(tab, src, dst, zeros, out, srcv, dstv, rows, acc,
          si0, si1, di0, di1, g0, g1, s0, s1):
        semis = (si0, si1)
        semid = (di0, di1)
        semg = (g0, g1)
        sems = (s0, s1)
        c = lax.axis_index("c")
        s = lax.axis_index("s")
        pltpu.sync_copy(zeros.at[pl.ds(s * share, share)],
                        acc.at[pl.ds(s * share, share)])
        plsc.subcore_barrier()
        tile_e0 = (c * 16 + s) * e_tile

        def issue_src(rr, st):
            for b in range(K):
                pltpu.async_copy(
                    src.at[pl.ds(tile_e0 + (rr * K + b) * _CH, _CH)],
                    srcv.at[st, b], semis[st])

        def issue_dst(rr, st):
            for b in range(K):
                pltpu.async_copy(
                    dst.at[pl.ds(tile_e0 + (rr * K + b) * _CH, _CH)],
                    dstv.at[st, b], semid[st])

        def drain_src(st):
            for b in range(K):
                pltpu.make_async_copy(src.at[pl.ds(0, _CH)],
                                      srcv.at[st, b], semis[st]).wait()

        def drain_dst(st):
            for b in range(K):
                pltpu.make_async_copy(dst.at[pl.ds(0, _CH)],
                                      dstv.at[st, b], semid[st]).wait()

        def issue_gathers(st):
            for b in range(K):
                pltpu.async_copy(tab.at[srcv.at[st, b]], rows.at[st, b],
                                 semg[st])

        def drain_gathers(st):
            for b in range(K):
                pltpu.make_async_copy(tab.at[pl.ds(0, _CH)], rows.at[st, b],
                                      semg[st]).wait()

        def issue_scatters(st):
            for b in range(K):
                pltpu.async_copy(rows.at[st, b], acc.at[dstv.at[st, b]],
                                 sems[st], add=True)

        def drain_scatters(st):
            for b in range(K):
                pltpu.make_async_copy(rows.at[st, b], acc.at[dstv.at[st, b]],
                                      sems[st]).wait()

        if rounds > 0:
            issue_src(0, 0)
            issue_dst(0, 0)
            if rounds > 1:
                issue_src(1, 1)
            drain_src(0)
            issue_gathers(0)

            def round_tpl(rr, st):
                ot = 1 - st
                drain_gathers(st)

                @pl.when(rr + 2 < rounds)
                def _():
                    issue_src(rr + 2, st)
                drain_dst(st)
                issue_scatters(st)

                @pl.when(rr >= 1)
                def _():
                    drain_scatters(ot)

                @pl.when(rr + 1 < rounds)
                def _():
                    drain_src(ot)
                    issue_gathers(ot)
                    issue_dst(rr + 1, ot)

            def body(i, carry):
                round_tpl(2 * i, 0)
                round_tpl(2 * i + 1, 1)
                return carry

            lax.fori_loop(0, pairs, body, 0)
            drain_scatters((rounds - 1) % 2)

        if tail > 0:
            t0 = tail if tail <= K else K
            t1 = tail - t0
            for b in range(t0):
                pltpu.sync_copy(
                    src.at[pl.ds(tile_e0 + (rounds * K + b) * _CH, _CH)],
                    srcv.at[0, b])
                pltpu.sync_copy(
                    dst.at[pl.ds(tile_e0 + (rounds * K + b) * _CH, _CH)],
                    dstv.at[0, b])
            for b in range(t1):
                pltpu.sync_copy(
                    src.at[pl.ds(tile_e0 + (rounds * K + t0 + b) * _CH, _CH)],
                    srcv.at[1, b])
                pltpu.sync_copy(
                    dst.at[pl.ds(tile_e0 + (rounds * K + t0 + b) * _CH, _CH)],
                    dstv.at[1, b])
            gds = [pltpu.async_copy(tab.at[srcv.at[0, b]], rows.at[0, b], g0)
                   for b in range(t0)]
            gds += [pltpu.async_copy(tab.at[srcv.at[1, b]], rows.at[1, b], g1)
                    for b in range(t1)]
            for d in gds:
                d.wait()
            sds = [pltpu.async_copy(rows.at[0, b], acc.at[dstv.at[0, b]],
                                    s0, add=True) for b in range(t0)]
            sds += [pltpu.async_copy(rows.at[1, b], acc.at[dstv.at[1, b]],
                                     s1, add=True) for b in range(t1)]
            for d in sds:
                d.wait()

        plsc.subcore_barrier()
        pltpu.sync_copy(acc.at[pl.ds(s * share, share)],
                        out.at[c, pl.ds(s * share, share)])

    return k


def _mm(h, W, R):
    n, Fi = h.shape
    Fo = W.shape[1]

    def body(h_ref, w_ref, o_ref):
        o_ref[...] = jnp.dot(h_ref[...], w_ref[...],
                             preferred_element_type=jnp.float32)

    return pl.pallas_call(
        body,
        grid=(n // R,),
        in_specs=[pl.BlockSpec((R, Fi), lambda i: (i, 0)),
                  pl.BlockSpec((Fi, Fo), lambda i: (0, 0))],
        out_specs=pl.BlockSpec((R, Fo), lambda i: (i, 0)),
        out_shape=jax.ShapeDtypeStruct((n, Fo), jnp.float32),
    )(h, W)


def _part_spec(arr, lead, R, F):
    if lead is None:
        return pl.BlockSpec((R, F), lambda i: (i, 0))
    return pl.BlockSpec((1, R, F), lambda i, l=lead: (l, i, 0))


def _part_val(ref):
    v = ref[...]
    return v[0] if v.ndim == 3 else v


def _stats(parts, n_real, R):
    F = parts[0][0].shape[-1]
    n = parts[0][0].shape[-2]
    nparts = len(parts)

    def body(*refs):
        i = pl.program_id(0)
        h = _part_val(refs[0])
        for r in refs[1:nparts]:
            h = h + _part_val(r)
        s1_ref, s2_ref = refs[-2], refs[-1]
        rows = lax.broadcasted_iota(jnp.int32, (R, F), 0) + i * R
        h = jnp.where(rows < n_real, h, 0.0)
        s1 = jnp.sum(h, axis=0, keepdims=True)
        s2 = jnp.sum(h * h, axis=0, keepdims=True)

        @pl.when(i == 0)
        def _():
            s1_ref[...] = jnp.zeros_like(s1_ref)
            s2_ref[...] = jnp.zeros_like(s2_ref)

        s1_ref[...] += s1
        s2_ref[...] += s2

    return pl.pallas_call(
        body,
        grid=(n // R,),
        in_specs=[_part_spec(a, l, R, F) for a, l in parts],
        out_specs=[pl.BlockSpec((1, F), lambda i: (0, 0)),
                   pl.BlockSpec((1, F), lambda i: (0, 0))],
        out_shape=[jax.ShapeDtypeStruct((1, F), jnp.float32),
                   jax.ShapeDtypeStruct((1, F), jnp.float32)],
    )(*[a for a, _ in parts])


def _bnapply(parts, s1, s2, g, b, n_real, R, res=None, Ws=(), head=None,
             want_act=False):
    F = parts[0][0].shape[-1]
    n = parts[0][0].shape[-2]
    n_parts = len(parts)
    has_res = res is not None
    has_head = head is not None

    def body(*refs):
        it = iter(refs)
        h = _part_val(next(it))
        for _ in range(n_parts - 1):
            h = h + _part_val(next(it))
        s1v = next(it)[...]
        s2v = next(it)[...]
        gv = next(it)[...]
        bv = next(it)[...]
        mu = s1v / n_real
        var = s2v / n_real - mu * mu
        inv = lax.rsqrt(var + 1e-5)
        t = jax.nn.relu((h - mu) * inv * gv + bv)
        res_v = next(it)[...] if has_res else None
        if has_head:
            wo = next(it)[...]
            bo = next(it)[...]
        w_refs = [next(it) for _ in Ws]
        outs = list(it)
        oi = 0
        if want_act:
            outs[oi][...] = t
            oi += 1
        if has_head:
            pv = jax.nn.sigmoid(
                jnp.dot(t, wo, preferred_element_type=jnp.float32) + bo)
            outs[oi][...] = pv[:, :1]
            oi += 1
        r = t + res_v if has_res else t
        for wr in w_refs:
            outs[oi][...] = jnp.dot(r, wr[...],
                                    preferred_element_type=jnp.float32)
            oi += 1

    in_arrays = [a for a, _ in parts] + [s1, s2, g, b]
    in_specs = [_part_spec(a, l, R, F) for a, l in parts]
    in_specs += [pl.BlockSpec((1, F), lambda i: (0, 0))] * 4
    if has_res:
        in_arrays.append(res)
        in_specs.append(pl.BlockSpec((R, F), lambda i: (i, 0)))
    if has_head:
        wo, bo = head
        in_arrays += [wo, bo]
        in_specs += [pl.BlockSpec(wo.shape, lambda i: (0, 0)),
                     pl.BlockSpec((1, 8), lambda i: (0, 0))]
    for W in Ws:
        in_arrays.append(W)
        in_specs.append(pl.BlockSpec(W.shape, lambda i: (0, 0)))

    out_specs, out_shapes = [], []
    if want_act:
        out_specs.append(pl.BlockSpec((R, F), lambda i: (i, 0)))
        out_shapes.append(jax.ShapeDtypeStruct((n, F), jnp.float32))
    if has_head:
        out_specs.append(pl.BlockSpec((R, 1), lambda i: (i, 0)))
        out_shapes.append(jax.ShapeDtypeStruct((n_real, 1), jnp.float32))
    for W in Ws:
        out_specs.append(pl.BlockSpec((R, W.shape[1]), lambda i: (i, 0)))
        out_shapes.append(jax.ShapeDtypeStruct((n, W.shape[1]), jnp.float32))

    return pl.pallas_call(
        body,
        grid=(n // R,),
        in_specs=in_specs,
        out_specs=out_specs,
        out_shape=out_shapes,
    )(*in_arrays)


def _final(P, dfin, wo1, bo1, R, n_real):
    n = P.shape[1]

    def body(p0_ref, p1_ref, d_ref, wo_ref, bo_ref, decf_ref, prob_ref):
        s = d_ref[...] + p0_ref[...][0] + p1_ref[...][0]
        decf = jax.nn.sigmoid(s)
        decf_ref[...] = decf[:, :3]
        pv = jax.nn.sigmoid(
            jnp.dot(decf, wo_ref[...], preferred_element_type=jnp.float32)
            + bo_ref[...])
        prob_ref[...] = pv[:, :1]

    return pl.pallas_call(
        body,
        grid=(n // R,),
        in_specs=[pl.BlockSpec((1, R, 16), lambda i: (0, i, 0)),
                  pl.BlockSpec((1, R, 16), lambda i: (1, i, 0)),
                  pl.BlockSpec((R, 16), lambda i: (i, 0)),
                  pl.BlockSpec((16, 8), lambda i: (0, 0)),
                  pl.BlockSpec((1, 8), lambda i: (0, 0))],
        out_specs=[pl.BlockSpec((R, 3), lambda i: (i, 0)),
                   pl.BlockSpec((R, 1), lambda i: (i, 0))],
        out_shape=[jax.ShapeDtypeStruct((n_real, 3), jnp.float32),
                   jax.ShapeDtypeStruct((n_real, 1), jnp.float32)],
    )(P, P, dfin, wo1, bo1)


def _bnstage(parts, g, b, n_real, R, res=None, Ws=(), head=None,
             want_act=False):
    n, F = parts[0].shape
    n_parts = len(parts)
    has_res = res is not None
    has_head = head is not None

    def body(*refs):
        stat = refs[-1]
        refs = refs[:-1]
        p = pl.program_id(0)
        i = pl.program_id(1)
        it = iter(refs)
        h = next(it)[...]
        for _ in range(n_parts - 1):
            h = h + next(it)[...]
        gv = next(it)[...]
        bv = next(it)[...]
        res_v = next(it)[...] if has_res else None
        if has_head:
            wo = next(it)[...]
            bo = next(it)[...]
        w_refs = [next(it) for _ in Ws]
        outs = list(it)

        @pl.when(p == 0)
        def _():
            rows = lax.broadcasted_iota(jnp.int32, (R, F), 0) + i * R
            hm = jnp.where(rows < n_real, h, 0.0)
            s1 = jnp.sum(hm, axis=0, keepdims=True)
            s2 = jnp.sum(hm * hm, axis=0, keepdims=True)

            @pl.when(i == 0)
            def _():
                stat[...] = jnp.zeros_like(stat)

            stat[0:1, :] += s1
            stat[1:2, :] += s2

        @pl.when(p == 1)
        def _():
            mu = stat[0:1, :] / n_real
            var = stat[1:2, :] / n_real - mu * mu
            inv = lax.rsqrt(var + 1e-5)
            t = jax.nn.relu((h - mu) * inv * gv + bv)
            oi = 0
            if want_act:
                outs[oi][...] = t
                oi += 1
            if has_head:
                outs[oi][...] = jax.nn.sigmoid(
                    jnp.dot(t, wo, preferred_element_type=jnp.float32) + bo)
                oi += 1
            r = t + res_v if has_res else t
            for wr in w_refs:
                outs[oi][...] = jnp.dot(r, wr[...],
                                        preferred_element_type=jnp.float32)
                oi += 1

    in_arrays = list(parts) + [g, b]
    in_specs = [pl.BlockSpec((R, F), lambda p, i: (i, 0)) for _ in parts]
    in_specs += [pl.BlockSpec((1, F), lambda p, i: (0, 0))] * 2
    if has_res:
        in_arrays.append(res)
        in_specs.append(pl.BlockSpec((R, F), lambda p, i: (i, 0)))
    if has_head:
        wo, bo = head
        in_arrays += [wo, bo]
        in_specs += [pl.BlockSpec(wo.shape, lambda p, i: (0, 0)),
                     pl.BlockSpec((1, 8), lambda p, i: (0, 0))]
    for W in Ws:
        in_arrays.append(W)
        in_specs.append(pl.BlockSpec(W.shape, lambda p, i: (0, 0)))

    out_specs, out_shapes = [], []
    if want_act:
        out_specs.append(pl.BlockSpec((R, F), lambda p, i: (i, 0)))
        out_shapes.append(jax.ShapeDtypeStruct((n, F), jnp.float32))
    if has_head:
        out_specs.append(pl.BlockSpec((R, 8), lambda p, i: (i, 0)))
        out_shapes.append(jax.ShapeDtypeStruct((n, 8), jnp.float32))
    for W in Ws:
        out_specs.append(pl.BlockSpec((R, W.shape[1]), lambda p, i: (i, 0)))
        out_shapes.append(jax.ShapeDtypeStruct((n, W.shape[1]), jnp.float32))

    return pl.pallas_call(
        body,
        grid=(2, n // R),
        in_specs=in_specs,
        out_specs=out_specs,
        out_shape=out_shapes,
        scratch_shapes=[pltpu.VMEM((2, F), jnp.float32)],
    )(*in_arrays)


def _mm2(h, W1, W2, R):
    n, Fi = h.shape

    def body(h_ref, w1_ref, w2_ref, o1_ref, o2_ref):
        hv = h_ref[...]
        o1_ref[...] = jnp.dot(hv, w1_ref[...],
                              preferred_element_type=jnp.float32)
        o2_ref[...] = jnp.dot(hv, w2_ref[...],
                              preferred_element_type=jnp.float32)

    return pl.pallas_call(
        body,
        grid=(n // R,),
        in_specs=[pl.BlockSpec((R, Fi), lambda i: (i, 0)),
                  pl.BlockSpec(W1.shape, lambda i: (0, 0)),
                  pl.BlockSpec(W2.shape, lambda i: (0, 0))],
        out_specs=[pl.BlockSpec((R, W1.shape[1]), lambda i: (i, 0)),
                   pl.BlockSpec((R, W2.shape[1]), lambda i: (i, 0))],
        out_shape=[jax.ShapeDtypeStruct((n, W1.shape[1]), jnp.float32),
                   jax.ShapeDtypeStruct((n, W2.shape[1]), jnp.float32)],
    )(h, W1, W2)


def _pad_edges(src, dst, n_pad_dst):
    e = src.shape[0]
    e_pad = -(-e // (_NTILE * _CH)) * (_NTILE * _CH)
    if e_pad != e:
        src = jnp.concatenate(
            [src, jnp.zeros((e_pad - e,), jnp.int32)])
        dst = jnp.concatenate(
            [dst, jnp.full((e_pad - e,), n_pad_dst - 1, jnp.int32)])
    return src, dst, e_pad


def _pad_w(W, cols):
    Fi, Fo = W.shape
    return jnp.pad(W, ((0, 0), (0, cols - Fo)))


def _row2d(v, width):
    v = jnp.asarray(v, jnp.float32).reshape(1, -1)
    return jnp.pad(v, ((0, 0), (0, width - v.shape[1])))


def kernel(x, ei0_src, ei0_dst, e2_src, e2_dst, e3_src, e3_dst, e4_src,
           e4_dst, params):
    p = params
    f32 = jnp.float32

    x_p = jnp.pad(x.astype(f32), ((0, _P0 - _N0), (0, 0)))
    z0_16 = jnp.zeros((_P0, 16), f32)
    z1_32 = jnp.zeros((_P1, 32), f32)
    z2_64 = jnp.zeros((_P2, 64), f32)
    z3_128 = jnp.zeros((_P3, 128), f32)

    ei_s, ei_d, E0p = _pad_edges(ei0_src, ei0_dst, _P0)
    e2_s, e2_d, E2p = _pad_edges(e2_src, e2_dst, _P1)
    e3_s, e3_d, E3p = _pad_edges(e3_src, e3_dst, _P2)
    e4_s, e4_d, E4p = _pad_edges(e4_src, e4_dst, _P3)
    e2r_s, e2r_d, _ = _pad_edges(e2_dst, e2_src, _P0)
    e3r_s, e3r_d, _ = _pad_edges(e3_dst, e3_src, _P1)
    e4r_s, e4r_d, _ = _pad_edges(e4_dst, e4_src, _P2)

    g1 = _row2d(p["g1"], 16); b1 = _row2d(p["b1"], 16)
    g2 = _row2d(p["g2"], 32); b2 = _row2d(p["b2"], 32)
    g3 = _row2d(p["g3"], 64); b3 = _row2d(p["b3"], 64)
    g4 = _row2d(p["g4"], 128); b4 = _row2d(p["b4"], 128)
    gd4 = _row2d(p["gd4"], 64); bd4 = _row2d(p["bd4"], 64)
    gd3 = _row2d(p["gd3"], 32); bd3 = _row2d(p["bd3"], 32)
    gd2 = _row2d(p["gd2"], 16); bd2 = _row2d(p["bd2"], 16)
    wo4 = _pad_w(p["wo4"], 8); bo4 = _row2d(p["bo4"], 8)
    wo3 = _pad_w(p["wo3"], 8); bo3 = _row2d(p["bo3"], 8)
    wo2 = _pad_w(p["wo2"], 8); bo2 = _row2d(p["bo2"], 8)
    wo1 = jnp.pad(_pad_w(p["wo1"], 8), ((0, 16 - 3), (0, 0)))
    bo1 = _row2d(p["bo1"], 8)
    wd1n = _pad_w(p["Wd1n"], 16)
    wd1s = _pad_w(p["Wd1s"], 16)

    tab0, d0 = _mm2(x_p, p["W1n"], p["W1s"], 512)
    P = _sconv_sc(_P0, 16, E0p)(tab0, ei_s, ei_d, z0_16)
    parts = [(P, 0), (P, 1), (d0, None)]
    s1, s2 = _stats(parts, _N0, 1024)
    enc0, tab1 = _bnapply(parts, s1, s2, g1, b1, _N0, 1024,
                          Ws=(p["W2"],), want_act=True)
    P = _sconv_sc(_P1, 32, E2p)(tab1, e2_s, e2_d, z1_32)
    parts = [(P, 0), (P, 1)]
    s1, s2 = _stats(parts, _N1, 512)
    enc1, tab2 = _bnapply(parts, s1, s2, g2, b2, _N1, 512,
                          Ws=(p["W3"],), want_act=True)
    P = _sconv_sc(_P2, 64, E3p)(tab2, e3_s, e3_d, z2_64)
    parts = [(P, 0), (P, 1)]
    s1, s2 = _stats(parts, _N2, 256)
    enc2, tab3 = _bnapply(parts, s1, s2, g3, b3, _N2, 256,
                          Ws=(p["W4"],), want_act=True)
    P = _sconv_sc(_P3, 128, E4p)(tab3, e4_s, e4_d, z3_128)
    parts = [(P, 0), (P, 1)]
    s1, s2 = _stats(parts, _N3, 128)
    tab4, = _bnapply(parts, s1, s2, g4, b4, _N3, 128, Ws=(p["Wd4"],))
    P = _sconv_sc(_P2, 64, E4p)(tab4, e4r_s, e4r_d, z2_64)
    parts = [(P, 0), (P, 1)]
    s1, s2 = _stats(parts, _N2, 256)
    prob4, tab5 = _bnapply(parts, s1, s2, gd4, bd4, _N2, 256,
                           res=enc2, Ws=(p["Wd3"],), head=(wo4, bo4))
    P = _sconv_sc(_P1, 32, E3p)(tab5, e3r_s, e3r_d, z1_32)
    parts = [(P, 0), (P, 1)]
    s1, s2 = _stats(parts, _N1, 512)
    prob3, tab6 = _bnapply(parts, s1, s2, gd3, bd3, _N1, 512,
                           res=enc1, Ws=(p["Wd2"],), head=(wo3, bo3))
    P = _sconv_sc(_P0, 16, E2p)(tab6, e2r_s, e2r_d, z0_16)
    parts = [(P, 0), (P, 1)]
    s1, s2 = _stats(parts, _N0, 1024)
    prob2, tab7, dfin = _bnapply(parts, s1, s2, gd2, bd2, _N0, 1024,
                                 res=enc0, Ws=(wd1n, wd1s),
                                 head=(wo2, bo2))
    P = _sconv_sc(_P0, 16, E0p)(tab7, ei_s, ei_d, z0_16)
    decf, prob1 = _final(P, dfin, wo1, bo1, 1024, _N0)

    return (decf, prob4, prob3, prob2, prob1)


# speedup vs baseline: 1.2431x; 1.0005x over previous
import functools

import jax
import jax.numpy as jnp
from jax import lax
from jax.experimental import pallas as pl
from jax.experimental.pallas import tpu as pltpu
from jax.experimental.pallas import tpu_sc as plsc

_N0, _N1, _N2, _N3 = 50000, 25000, 12500, 6250
_P0, _P1, _P2, _P3 = 50176, 25088, 12544, 6272

_CH = 128
_N_BY_PAD = {_P0: _N0, _P1: _N1, _P2: _N2, _P3: _N3}
_NTILE = 32


def _sconv_sc(n_pad, F, E_pad):
    e_tile = E_pad // _NTILE
    chunks = e_tile // _CH
    K = {16: 16, 32: 8, 64: 4, 128: 2}[F]
    rounds = chunks // K
    rounds -= rounds % 2
    tail = chunks - rounds * K
    assert tail < 2 * K
    pairs = rounds // 2
    share = n_pad // 16
    mesh = plsc.VectorSubcoreMesh(core_axis_name="c", subcore_axis_name="s")

    @functools.partial(
        pl.kernel,
        out_type=jax.ShapeDtypeStruct((2, n_pad, F), jnp.float32),
        mesh=mesh,
        compiler_params=pltpu.CompilerParams(use_tc_tiling_on_sc=False),
        scratch_types=[
            pltpu.VMEM((2, K, _CH), jnp.int32),
            pltpu.VMEM((2, K, _CH), jnp.int32),
            pltpu.VMEM((2, K, _CH, F), jnp.float32),
            pltpu.VMEM_SHARED((n_pad, F), jnp.float32),
            pltpu.SemaphoreType.DMA,
            pltpu.SemaphoreType.DMA,
            pltpu.SemaphoreType.DMA,
            pltpu.SemaphoreType.DMA,
            pltpu.SemaphoreType.DMA,
            pltpu.SemaphoreType.DMA,
            pltpu.SemaphoreType.DMA,
            pltpu.SemaphoreType.DMA,
        ],
    )
    def k(tab, src, dst, zeros, out, srcv, dstv, rows, acc,
          si0, si1, di0, di1, g0, g1, s0, s1):
        semis = (si0, si1)
        semid = (di0, di1)
        semg = (g0, g1)
        sems = (s0, s1)
        c = lax.axis_index("c")
        s = lax.axis_index("s")
        pltpu.sync_copy(zeros.at[pl.ds(s * share, share)],
                        acc.at[pl.ds(s * share, share)])
        plsc.subcore_barrier()
        tile_e0 = (c * 16 + s) * e_tile

        def issue_src(rr, st):
            for b in range(K):
                pltpu.async_copy(
                    src.at[pl.ds(tile_e0 + (rr * K + b) * _CH, _CH)],
                    srcv.at[st, b], semis[st])

        def issue_dst(rr, st):
            for b in range(K):
                pltpu.async_copy(
                    dst.at[pl.ds(tile_e0 + (rr * K + b) * _CH, _CH)],
                    dstv.at[st, b], semid[st])

        def drain_src(st):
            for b in range(K):
                pltpu.make_async_copy(src.at[pl.ds(0, _CH)],
                                      srcv.at[st, b], semis[st]).wait()

        def drain_dst(st):
            for b in range(K):
                pltpu.make_async_copy(dst.at[pl.ds(0, _CH)],
                                      dstv.at[st, b], semid[st]).wait()

        def issue_gathers(st):
            for b in range(K):
                pltpu.async_copy(tab.at[srcv.at[st, b]], rows.at[st, b],
                                 semg[st])

        def drain_gathers(st):
            for b in range(K):
                pltpu.make_async_copy(tab.at[pl.ds(0, _CH)], rows.at[st, b],
                                      semg[st]).wait()

        def issue_scatters(st):
            for b in range(K):
                pltpu.async_copy(rows.at[st, b], acc.at[dstv.at[st, b]],
                                 sems[st], add=True)

        def drain_scatters(st):
            for b in range(K):
                pltpu.make_async_copy(rows.at[st, b], acc.at[dstv.at[st, b]],
                                      sems[st]).wait()

        if rounds > 0:
            issue_src(0, 0)
            issue_dst(0, 0)
            if rounds > 1:
                issue_src(1, 1)
            drain_src(0)
            issue_gathers(0)

            def round_tpl(rr, st):
                ot = 1 - st
                drain_gathers(st)

                @pl.when(rr + 2 < rounds)
                def _():
                    issue_src(rr + 2, st)
                drain_dst(st)
                issue_scatters(st)

                @pl.when(rr >= 1)
                def _():
                    drain_scatters(ot)

                @pl.when(rr + 1 < rounds)
                def _():
                    drain_src(ot)
                    issue_gathers(ot)
                    issue_dst(rr + 1, ot)

            def body(i, carry):
                round_tpl(2 * i, 0)
                round_tpl(2 * i + 1, 1)
                return carry

            lax.fori_loop(0, pairs, body, 0)
            drain_scatters((rounds - 1) % 2)

        if tail > 0:
            t0 = tail if tail <= K else K
            t1 = tail - t0
            for b in range(t0):
                pltpu.sync_copy(
                    src.at[pl.ds(tile_e0 + (rounds * K + b) * _CH, _CH)],
                    srcv.at[0, b])
                pltpu.sync_copy(
                    dst.at[pl.ds(tile_e0 + (rounds * K + b) * _CH, _CH)],
                    dstv.at[0, b])
            for b in range(t1):
                pltpu.sync_copy(
                    src.at[pl.ds(tile_e0 + (rounds * K + t0 + b) * _CH, _CH)],
                    srcv.at[1, b])
                pltpu.sync_copy(
                    dst.at[pl.ds(tile_e0 + (rounds * K + t0 + b) * _CH, _CH)],
                    dstv.at[1, b])
            gds = [pltpu.async_copy(tab.at[srcv.at[0, b]], rows.at[0, b], g0)
                   for b in range(t0)]
            gds += [pltpu.async_copy(tab.at[srcv.at[1, b]], rows.at[1, b], g1)
                    for b in range(t1)]
            for d in gds:
                d.wait()
            sds = [pltpu.async_copy(rows.at[0, b], acc.at[dstv.at[0, b]],
                                    s0, add=True) for b in range(t0)]
            sds += [pltpu.async_copy(rows.at[1, b], acc.at[dstv.at[1, b]],
                                     s1, add=True) for b in range(t1)]
            for d in sds:
                d.wait()

        plsc.subcore_barrier()
        pltpu.sync_copy(acc.at[pl.ds(s * share, share)],
                        out.at[c, pl.ds(s * share, share)])

    return k


def _mm(h, W, R):
    n, Fi = h.shape
    Fo = W.shape[1]

    def body(h_ref, w_ref, o_ref):
        o_ref[...] = jnp.dot(h_ref[...], w_ref[...],
                             preferred_element_type=jnp.float32)

    return pl.pallas_call(
        body,
        grid=(n // R,),
        in_specs=[pl.BlockSpec((R, Fi), lambda i: (i, 0)),
                  pl.BlockSpec((Fi, Fo), lambda i: (0, 0))],
        out_specs=pl.BlockSpec((R, Fo), lambda i: (i, 0)),
        out_shape=jax.ShapeDtypeStruct((n, Fo), jnp.float32),
    )(h, W)


def _part_spec(arr, lead, R, F):
    if lead is None:
        return pl.BlockSpec((R, F), lambda i: (i, 0))
    return pl.BlockSpec((1, R, F), lambda i, l=lead: (l, i, 0))


def _part_val(ref):
    v = ref[...]
    return v[0] if v.ndim == 3 else v


def _stats(parts, n_real, R):
    F = parts[0][0].shape[-1]
    n = parts[0][0].shape[-2]
    nparts = len(parts)

    def body(*refs):
        i = pl.program_id(0)
        h = _part_val(refs[0])
        for r in refs[1:nparts]:
            h = h + _part_val(r)
        s1_ref, s2_ref = refs[-2], refs[-1]
        rows = lax.broadcasted_iota(jnp.int32, (R, F), 0) + i * R
        h = jnp.where(rows < n_real, h, 0.0)
        s1 = jnp.sum(h, axis=0, keepdims=True)
        s2 = jnp.sum(h * h, axis=0, keepdims=True)

        @pl.when(i == 0)
        def _():
            s1_ref[...] = jnp.zeros_like(s1_ref)
            s2_ref[...] = jnp.zeros_like(s2_ref)

        s1_ref[...] += s1
        s2_ref[...] += s2

    return pl.pallas_call(
        body,
        grid=(n // R,),
        in_specs=[_part_spec(a, l, R, F) for a, l in parts],
        out_specs=[pl.BlockSpec((1, F), lambda i: (0, 0)),
                   pl.BlockSpec((1, F), lambda i: (0, 0))],
        out_shape=[jax.ShapeDtypeStruct((1, F), jnp.float32),
                   jax.ShapeDtypeStruct((1, F), jnp.float32)],
    )(*[a for a, _ in parts])


def _bnapply(parts, s1, s2, g, b, n_real, R, res=None, Ws=(), head=None,
             want_act=False):
    F = parts[0][0].shape[-1]
    n = parts[0][0].shape[-2]
    n_parts = len(parts)
    has_res = res is not None
    has_head = head is not None

    def body(*refs):
        it = iter(refs)
        h = _part_val(next(it))
        for _ in range(n_parts - 1):
            h = h + _part_val(next(it))
        s1v = next(it)[...]
        s2v = next(it)[...]
        gv = next(it)[...]
        bv = next(it)[...]
        mu = s1v / n_real
        var = s2v / n_real - mu * mu
        inv = lax.rsqrt(var + 1e-5)
        t = jax.nn.relu((h - mu) * inv * gv + bv)
        res_v = next(it)[...] if has_res else None
        if has_head:
            wo = next(it)[...]
            bo = next(it)[...]
        w_refs = [next(it) for _ in Ws]
        outs = list(it)
        oi = 0
        if want_act:
            outs[oi][...] = t
            oi += 1
        if has_head:
            pv = jax.nn.sigmoid(
                jnp.dot(t, wo, preferred_element_type=jnp.float32) + bo)
            outs[oi][...] = pv[:, :1]
            oi += 1
        r = t + res_v if has_res else t
        for wr in w_refs:
            outs[oi][...] = jnp.dot(r, wr[...],
                                    preferred_element_type=jnp.float32)
            oi += 1

    in_arrays = [a for a, _ in parts] + [s1, s2, g, b]
    in_specs = [_part_spec(a, l, R, F) for a, l in parts]
    in_specs += [pl.BlockSpec((1, F), lambda i: (0, 0))] * 4
    if has_res:
        in_arrays.append(res)
        in_specs.append(pl.BlockSpec((R, F), lambda i: (i, 0)))
    if has_head:
        wo, bo = head
        in_arrays += [wo, bo]
        in_specs += [pl.BlockSpec(wo.shape, lambda i: (0, 0)),
                     pl.BlockSpec((1, 8), lambda i: (0, 0))]
    for W in Ws:
        in_arrays.append(W)
        in_specs.append(pl.BlockSpec(W.shape, lambda i: (0, 0)))

    out_specs, out_shapes = [], []
    if want_act:
        out_specs.append(pl.BlockSpec((R, F), lambda i: (i, 0)))
        out_shapes.append(jax.ShapeDtypeStruct((n, F), jnp.float32))
    if has_head:
        out_specs.append(pl.BlockSpec((R, 1), lambda i: (i, 0)))
        out_shapes.append(jax.ShapeDtypeStruct((n_real, 1), jnp.float32))
    for W in Ws:
        out_specs.append(pl.BlockSpec((R, W.shape[1]), lambda i: (i, 0)))
        out_shapes.append(jax.ShapeDtypeStruct((n, W.shape[1]), jnp.float32))

    return pl.pallas_call(
        body,
        grid=(n // R,),
        in_specs=in_specs,
        out_specs=out_specs,
        out_shape=out_shapes,
    )(*in_arrays)


def _final(P, dfin, wo1, bo1, R, n_real):
    n = P.shape[1]

    def body(p0_ref, p1_ref, d_ref, wo_ref, bo_ref, decf_ref, prob_ref):
        s = d_ref[...] + p0_ref[...][0] + p1_ref[...][0]
        decf = jax.nn.sigmoid(s)
        decf_ref[...] = decf[:, :3]
        pv = jax.nn.sigmoid(
            jnp.dot(decf, wo_ref[...], preferred_element_type=jnp.float32)
            + bo_ref[...])
        prob_ref[...] = pv[:, :1]

    return pl.pallas_call(
        body,
        grid=(n // R,),
        in_specs=[pl.BlockSpec((1, R, 16), lambda i: (0, i, 0)),
                  pl.BlockSpec((1, R, 16), lambda i: (1, i, 0)),
                  pl.BlockSpec((R, 16), lambda i: (i, 0)),
                  pl.BlockSpec((16, 8), lambda i: (0, 0)),
                  pl.BlockSpec((1, 8), lambda i: (0, 0))],
        out_specs=[pl.BlockSpec((R, 3), lambda i: (i, 0)),
                   pl.BlockSpec((R, 1), lambda i: (i, 0))],
        out_shape=[jax.ShapeDtypeStruct((n_real, 3), jnp.float32),
                   jax.ShapeDtypeStruct((n_real, 1), jnp.float32)],
    )(P, P, dfin, wo1, bo1)


def _bnstage(parts, g, b, n_real, R, res=None, Ws=(), head=None,
             want_act=False):
    n, F = parts[0].shape
    n_parts = len(parts)
    has_res = res is not None
    has_head = head is not None

    def body(*refs):
        stat = refs[-1]
        refs = refs[:-1]
        p = pl.program_id(0)
        i = pl.program_id(1)
        it = iter(refs)
        h = next(it)[...]
        for _ in range(n_parts - 1):
            h = h + next(it)[...]
        gv = next(it)[...]
        bv = next(it)[...]
        res_v = next(it)[...] if has_res else None
        if has_head:
            wo = next(it)[...]
            bo = next(it)[...]
        w_refs = [next(it) for _ in Ws]
        outs = list(it)

        @pl.when(p == 0)
        def _():
            rows = lax.broadcasted_iota(jnp.int32, (R, F), 0) + i * R
            hm = jnp.where(rows < n_real, h, 0.0)
            s1 = jnp.sum(hm, axis=0, keepdims=True)
            s2 = jnp.sum(hm * hm, axis=0, keepdims=True)

            @pl.when(i == 0)
            def _():
                stat[...] = jnp.zeros_like(stat)

            stat[0:1, :] += s1
            stat[1:2, :] += s2

        @pl.when(p == 1)
        def _():
            mu = stat[0:1, :] / n_real
            var = stat[1:2, :] / n_real - mu * mu
            inv = lax.rsqrt(var + 1e-5)
            t = jax.nn.relu((h - mu) * inv * gv + bv)
            oi = 0
            if want_act:
                outs[oi][...] = t
                oi += 1
            if has_head:
                outs[oi][...] = jax.nn.sigmoid(
                    jnp.dot(t, wo, preferred_element_type=jnp.float32) + bo)
                oi += 1
            r = t + res_v if has_res else t
            for wr in w_refs:
                outs[oi][...] = jnp.dot(r, wr[...],
                                        preferred_element_type=jnp.float32)
                oi += 1

    in_arrays = list(parts) + [g, b]
    in_specs = [pl.BlockSpec((R, F), lambda p, i: (i, 0)) for _ in parts]
    in_specs += [pl.BlockSpec((1, F), lambda p, i: (0, 0))] * 2
    if has_res:
        in_arrays.append(res)
        in_specs.append(pl.BlockSpec((R, F), lambda p, i: (i, 0)))
    if has_head:
        wo, bo = head
        in_arrays += [wo, bo]
        in_specs += [pl.BlockSpec(wo.shape, lambda p, i: (0, 0)),
                     pl.BlockSpec((1, 8), lambda p, i: (0, 0))]
    for W in Ws:
        in_arrays.append(W)
        in_specs.append(pl.BlockSpec(W.shape, lambda p, i: (0, 0)))

    out_specs, out_shapes = [], []
    if want_act:
        out_specs.append(pl.BlockSpec((R, F), lambda p, i: (i, 0)))
        out_shapes.append(jax.ShapeDtypeStruct((n, F), jnp.float32))
    if has_head:
        out_specs.append(pl.BlockSpec((R, 8), lambda p, i: (i, 0)))
        out_shapes.append(jax.ShapeDtypeStruct((n, 8), jnp.float32))
    for W in Ws:
        out_specs.append(pl.BlockSpec((R, W.shape[1]), lambda p, i: (i, 0)))
        out_shapes.append(jax.ShapeDtypeStruct((n, W.shape[1]), jnp.float32))

    return pl.pallas_call(
        body,
        grid=(2, n // R),
        in_specs=in_specs,
        out_specs=out_specs,
        out_shape=out_shapes,
        scratch_shapes=[pltpu.VMEM((2, F), jnp.float32)],
    )(*in_arrays)


def _mm2(h, W1, W2, R):
    n, Fi = h.shape

    def body(h_ref, w1_ref, w2_ref, o1_ref, o2_ref):
        hv = h_ref[...]
        o1_ref[...] = jnp.dot(hv, w1_ref[...],
                              preferred_element_type=jnp.float32)
        o2_ref[...] = jnp.dot(hv, w2_ref[...],
                              preferred_element_type=jnp.float32)

    return pl.pallas_call(
        body,
        grid=(n // R,),
        in_specs=[pl.BlockSpec((R, Fi), lambda i: (i, 0)),
                  pl.BlockSpec(W1.shape, lambda i: (0, 0)),
                  pl.BlockSpec(W2.shape, lambda i: (0, 0))],
        out_specs=[pl.BlockSpec((R, W1.shape[1]), lambda i: (i, 0)),
                   pl.BlockSpec((R, W2.shape[1]), lambda i: (i, 0))],
        out_shape=[jax.ShapeDtypeStruct((n, W1.shape[1]), jnp.float32),
                   jax.ShapeDtypeStruct((n, W2.shape[1]), jnp.float32)],
    )(h, W1, W2)


def _pad_edges(src, dst, n_pad_dst):
    e = src.shape[0]
    e_pad = -(-e // (_NTILE * _CH)) * (_NTILE * _CH)
    if e_pad != e:
        npad = e_pad - e
        src = jnp.concatenate([src, jnp.zeros((npad,), jnp.int32)])
        spread = _N_BY_PAD[n_pad_dst] + (
            jnp.arange(npad, dtype=jnp.int32) % (n_pad_dst - _N_BY_PAD[n_pad_dst]))
        dst = jnp.concatenate([dst, spread])
    return src, dst, e_pad


def _pad_w(W, cols):
    Fi, Fo = W.shape
    return jnp.pad(W, ((0, 0), (0, cols - Fo)))


def _row2d(v, width):
    v = jnp.asarray(v, jnp.float32).reshape(1, -1)
    return jnp.pad(v, ((0, 0), (0, width - v.shape[1])))


def kernel(x, ei0_src, ei0_dst, e2_src, e2_dst, e3_src, e3_dst, e4_src,
           e4_dst, params):
    p = params
    f32 = jnp.float32

    x_p = jnp.pad(x.astype(f32), ((0, _P0 - _N0), (0, 0)))
    z0_16 = jnp.zeros((_P0, 16), f32)
    z1_32 = jnp.zeros((_P1, 32), f32)
    z2_64 = jnp.zeros((_P2, 64), f32)
    z3_128 = jnp.zeros((_P3, 128), f32)

    ei_s, ei_d, E0p = _pad_edges(ei0_src, ei0_dst, _P0)
    e2_s, e2_d, E2p = _pad_edges(e2_src, e2_dst, _P1)
    e3_s, e3_d, E3p = _pad_edges(e3_src, e3_dst, _P2)
    e4_s, e4_d, E4p = _pad_edges(e4_src, e4_dst, _P3)
    e2r_s, e2r_d, _ = _pad_edges(e2_dst, e2_src, _P0)
    e3r_s, e3r_d, _ = _pad_edges(e3_dst, e3_src, _P1)
    e4r_s, e4r_d, _ = _pad_edges(e4_dst, e4_src, _P2)

    g1 = _row2d(p["g1"], 16); b1 = _row2d(p["b1"], 16)
    g2 = _row2d(p["g2"], 32); b2 = _row2d(p["b2"], 32)
    g3 = _row2d(p["g3"], 64); b3 = _row2d(p["b3"], 64)
    g4 = _row2d(p["g4"], 128); b4 = _row2d(p["b4"], 128)
    gd4 = _row2d(p["gd4"], 64); bd4 = _row2d(p["bd4"], 64)
    gd3 = _row2d(p["gd3"], 32); bd3 = _row2d(p["bd3"], 32)
    gd2 = _row2d(p["gd2"], 16); bd2 = _row2d(p["bd2"], 16)
    wo4 = _pad_w(p["wo4"], 8); bo4 = _row2d(p["bo4"], 8)
    wo3 = _pad_w(p["wo3"], 8); bo3 = _row2d(p["bo3"], 8)
    wo2 = _pad_w(p["wo2"], 8); bo2 = _row2d(p["bo2"], 8)
    wo1 = jnp.pad(_pad_w(p["wo1"], 8), ((0, 16 - 3), (0, 0)))
    bo1 = _row2d(p["bo1"], 8)
    wd1n = _pad_w(p["Wd1n"], 16)
    wd1s = _pad_w(p["Wd1s"], 16)

    tab0, d0 = _mm2(x_p, p["W1n"], p["W1s"], 512)
    P = _sconv_sc(_P0, 16, E0p)(tab0, ei_s, ei_d, z0_16)
    parts = [(P, 0), (P, 1), (d0, None)]
    s1, s2 = _stats(parts, _N0, 1024)
    enc0, tab1 = _bnapply(parts, s1, s2, g1, b1, _N0, 1024,
                          Ws=(p["W2"],), want_act=True)
    P = _sconv_sc(_P1, 32, E2p)(tab1, e2_s, e2_d, z1_32)
    parts = [(P, 0), (P, 1)]
    s1, s2 = _stats(parts, _N1, 512)
    enc1, tab2 = _bnapply(parts, s1, s2, g2, b2, _N1, 512,
                          Ws=(p["W3"],), want_act=True)
    P = _sconv_sc(_P2, 64, E3p)(tab2, e3_s, e3_d, z2_64)
    parts = [(P, 0), (P, 1)]
    s1, s2 = _stats(parts, _N2, 256)
    enc2, tab3 = _bnapply(parts, s1, s2, g3, b3, _N2, 256,
                          Ws=(p["W4"],), want_act=True)
    P = _sconv_sc(_P3, 128, E4p)(tab3, e4_s, e4_d, z3_128)
    parts = [(P, 0), (P, 1)]
    s1, s2 = _stats(parts, _N3, 128)
    tab4, = _bnapply(parts, s1, s2, g4, b4, _N3, 128, Ws=(p["Wd4"],))
    P = _sconv_sc(_P2, 64, E4p)(tab4, e4r_s, e4r_d, z2_64)
    parts = [(P, 0), (P, 1)]
    s1, s2 = _stats(parts, _N2, 256)
    prob4, tab5 = _bnapply(parts, s1, s2, gd4, bd4, _N2, 256,
                           res=enc2, Ws=(p["Wd3"],), head=(wo4, bo4))
    P = _sconv_sc(_P1, 32, E3p)(tab5, e3r_s, e3r_d, z1_32)
    parts = [(P, 0), (P, 1)]
    s1, s2 = _stats(parts, _N1, 512)
    prob3, tab6 = _bnapply(parts, s1, s2, gd3, bd3, _N1, 512,
                           res=enc1, Ws=(p["Wd2"],), head=(wo3, bo3))
    P = _sconv_sc(_P0, 16, E2p)(tab6, e2r_s, e2r_d, z0_16)
    parts = [(P, 0), (P, 1)]
    s1, s2 = _stats(parts, _N0, 1024)
    prob2, tab7, dfin = _bnapply(parts, s1, s2, gd2, bd2, _N0, 1024,
                                 res=enc0, Ws=(wd1n, wd1s),
                                 head=(wo2, bo2))
    P = _sconv_sc(_P0, 16, E0p)(tab7, ei_s, ei_d, z0_16)
    decf, prob1 = _final(P, dfin, wo1, bo1, 1024, _N0)

    return (decf, prob4, prob3, prob2, prob1)
